# Initial kernel scaffold; baseline (speedup 1.0000x reference)
import jax
import jax.numpy as jnp
from jax.experimental import pallas as pl


def kernel(x, adj, W1, b1, W2, b2):
    raise NotImplementedError("write your pallas kernel here")


# speedup vs baseline: 13.4776x; 13.4776x over previous
import functools

import jax
import jax.numpy as jnp
from jax import lax
from jax.experimental import pallas as pl
from jax.experimental.pallas import tpu as pltpu
from jax.experimental.pallas import tpu_sc as plsc

_NC = 2
_NS = 16
_L = 16
_NW = _NC * _NS
_CH = 80


def _mesh():
    return plsc.VectorSubcoreMesh(
        core_axis_name="c", subcore_axis_name="s",
        num_cores=_NC, num_subcores=_NS)


_SC_PARAMS = pltpu.CompilerParams(use_tc_tiling_on_sc=False)


def _sc_degree(dst, n_nodes):
    e = dst.shape[0]
    epw = e // _NW
    nch = epw // _CH
    rpt = n_nodes // _NS

    @functools.partial(
        pl.kernel,
        out_type=jax.ShapeDtypeStruct((_NC, n_nodes, _L), jnp.float32),
        mesh=_mesh(),
        scratch_types=[
            pltpu.VMEM((_CH,), jnp.int32),
            pltpu.VMEM((_CH, _L), jnp.float32),
            pltpu.VMEM((rpt, _L), jnp.float32),
            pltpu.VMEM_SHARED((n_nodes, _L), jnp.float32),
        ],
        compiler_params=_SC_PARAMS,
    )
    def k(dst_hbm, out_hbm, didx, ones_v, zb, acc):
        cid = lax.axis_index("c")
        sid = lax.axis_index("s")
        wid = sid * _NC + cid
        one = jnp.ones((_L,), jnp.float32)
        zero = jnp.zeros((_L,), jnp.float32)

        @pl.loop(0, _CH)
        def _(r):
            ones_v[r, :] = one

        @pl.loop(0, rpt)
        def _(r):
            zb[r, :] = zero

        row0 = sid * rpt
        pltpu.sync_copy(zb, acc.at[pl.ds(row0, rpt)])
        plsc.subcore_barrier()

        @pl.loop(0, nch)
        def _(c):
            off = pl.multiple_of(wid * epw + c * _CH, 8)
            pltpu.sync_copy(dst_hbm.at[pl.ds(off, _CH)], didx)
            pltpu.sync_copy(ones_v, acc.at[didx], add=True)

        plsc.subcore_barrier()
        pltpu.sync_copy(acc.at[pl.ds(row0, rpt)],
                        out_hbm.at[cid, pl.ds(row0, rpt)])

    return k(dst)


def _sc_aggregate(xs, src, dst, n_nodes):
    e = src.shape[0]
    d = xs.shape[1]
    epw = e // _NW
    nch = epw // _CH
    rpt = n_nodes // _NS
    zr = 125

    @functools.partial(
        pl.kernel,
        out_type=jax.ShapeDtypeStruct((_NC, n_nodes, d), jnp.float32),
        mesh=_mesh(),
        scratch_types=[
            pltpu.VMEM((_CH,), jnp.int32),
            pltpu.VMEM((_CH,), jnp.int32),
            pltpu.VMEM((_CH, d), jnp.float32),
            pltpu.VMEM((zr, d), jnp.float32),
            pltpu.VMEM_SHARED((n_nodes, d), jnp.float32),
            pltpu.SemaphoreType.DMA,
        ],
        compiler_params=_SC_PARAMS,
    )
    def k(xs_hbm, src_hbm, dst_hbm, out_hbm, sidx, didx, rows, zb, acc, sem):
        cid = lax.axis_index("c")
        sid = lax.axis_index("s")
        wid = sid * _NC + cid
        zero = jnp.zeros((_L,), jnp.float32)

        @pl.loop(0, zr)
        def _(r):
            @pl.loop(0, d // _L)
            def _(j):
                zb[r, pl.ds(j * _L, _L)] = zero

        row0 = sid * rpt

        @pl.loop(0, rpt // zr)
        def _(t):
            pltpu.sync_copy(zb, acc.at[pl.ds(row0 + t * zr, zr)])

        plsc.subcore_barrier()

        @pl.loop(0, nch)
        def _(c):
            off = pl.multiple_of(wid * epw + c * _CH, 8)
            pltpu.sync_copy(src_hbm.at[pl.ds(off, _CH)], sidx)
            pltpu.sync_copy(dst_hbm.at[pl.ds(off, _CH)], didx)
            pltpu.async_copy(xs_hbm.at[sidx], rows, sem).wait()
            pltpu.sync_copy(rows, acc.at[didx], add=True)

        plsc.subcore_barrier()

        @pl.loop(0, rpt // zr)
        def _(t):
            r0 = row0 + t * zr
            pltpu.sync_copy(acc.at[pl.ds(r0, zr)],
                            out_hbm.at[cid, pl.ds(r0, zr)])

    return k(xs, src, dst)


_RB = 1000


def _tc_prep1(x, w1, degp):
    n, d = x.shape

    def body(x_ref, w_ref, degp_ref, xs_ref, dinv_ref):
        deg = degp_ref[0, :, 0:1] + degp_ref[1, :, 0:1] + 1.0
        dinv = lax.rsqrt(deg)
        xw = jnp.dot(x_ref[...], w_ref[...],
                     preferred_element_type=jnp.float32)
        xs_ref[...] = xw * dinv
        dinv_ref[...] = dinv

    return pl.pallas_call(
        body,
        grid=(n // _RB,),
        in_specs=[
            pl.BlockSpec((_RB, d), lambda i: (i, 0)),
            pl.BlockSpec((d, d), lambda i: (0, 0)),
            pl.BlockSpec((_NC, _RB, _L), lambda i: (0, i, 0)),
        ],
        out_specs=[
            pl.BlockSpec((_RB, d), lambda i: (i, 0)),
            pl.BlockSpec((_RB, 1), lambda i: (i, 0)),
        ],
        out_shape=[
            jax.ShapeDtypeStruct((n, d), jnp.float32),
            jax.ShapeDtypeStruct((n, 1), jnp.float32),
        ],
    )(x, w1, degp)


def _tc_mid(agg, xs1, dinv, b1, w2):
    n, d = xs1.shape

    def body(agg_ref, xs_ref, dinv_ref, b_ref, w_ref, o_ref):
        s = agg_ref[0] + agg_ref[1] + xs_ref[...]
        h = jnp.maximum(dinv_ref[...] * s + b_ref[...], 0.0)
        o_ref[...] = jnp.dot(h, w_ref[...],
                             preferred_element_type=jnp.float32) * dinv_ref[...]

    return pl.pallas_call(
        body,
        grid=(n // _RB,),
        in_specs=[
            pl.BlockSpec((_NC, _RB, d), lambda i: (0, i, 0)),
            pl.BlockSpec((_RB, d), lambda i: (i, 0)),
            pl.BlockSpec((_RB, 1), lambda i: (i, 0)),
            pl.BlockSpec((1, d), lambda i: (0, 0)),
            pl.BlockSpec((d, d), lambda i: (0, 0)),
        ],
        out_specs=pl.BlockSpec((_RB, d), lambda i: (i, 0)),
        out_shape=jax.ShapeDtypeStruct((n, d), jnp.float32),
    )(agg, xs1, dinv, b1, w2)


def _tc_final(agg, xs2, dinv, b2):
    n, d = xs2.shape

    def body(agg_ref, xs_ref, dinv_ref, b_ref, o_ref):
        s = agg_ref[0] + agg_ref[1] + xs_ref[...]
        o_ref[...] = dinv_ref[...] * s + b_ref[...]

    return pl.pallas_call(
        body,
        grid=(n // _RB,),
        in_specs=[
            pl.BlockSpec((_NC, _RB, d), lambda i: (0, i, 0)),
            pl.BlockSpec((_RB, d), lambda i: (i, 0)),
            pl.BlockSpec((_RB, 1), lambda i: (i, 0)),
            pl.BlockSpec((1, d), lambda i: (0, 0)),
        ],
        out_specs=pl.BlockSpec((_RB, d), lambda i: (i, 0)),
        out_shape=jax.ShapeDtypeStruct((n, d), jnp.float32),
    )(agg, xs2, dinv, b2)


def kernel(x, adj, W1, b1, W2, b2):
    n, d = x.shape
    adj = adj.astype(jnp.int32)
    src = adj[0]
    dst = adj[1]
    e = src.shape[0]
    assert e % (_NW * _CH) == 0 and n % _NS == 0 and d % _L == 0

    degp = _sc_degree(dst, n)
    xs1, dinv = _tc_prep1(x, W1, degp)
    agg1 = _sc_aggregate(xs1, src, dst, n)
    xs2 = _tc_mid(agg1, xs1, dinv, b1.reshape(1, d), W2)
    agg2 = _sc_aggregate(xs2, src, dst, n)
    return _tc_final(agg2, xs2, dinv, b2.reshape(1, d))


# speedup vs baseline: 22.5463x; 1.6729x over previous
import functools

import jax
import jax.numpy as jnp
from jax import lax
from jax.experimental import pallas as pl
from jax.experimental.pallas import tpu as pltpu
from jax.experimental.pallas import tpu_sc as plsc

_NC = 2
_NS = 16
_L = 16
_NW = _NC * _NS
_CH = 40


def _mesh():
    return plsc.VectorSubcoreMesh(
        core_axis_name="c", subcore_axis_name="s",
        num_cores=_NC, num_subcores=_NS)


_SC_PARAMS = pltpu.CompilerParams(use_tc_tiling_on_sc=False)


def _sc_degree(dst, n_nodes):
    e = dst.shape[0]
    epw = e // _NW
    nch = epw // _CH
    rpt = n_nodes // _NS
    nb = 2
    nr = nch // nb

    @functools.partial(
        pl.kernel,
        out_type=jax.ShapeDtypeStruct((_NC, n_nodes, _L), jnp.float32),
        mesh=_mesh(),
        scratch_types=[
            pltpu.VMEM((_CH,), jnp.int32),
            pltpu.VMEM((_CH,), jnp.int32),
            pltpu.VMEM((_CH, _L), jnp.float32),
            pltpu.VMEM((rpt, _L), jnp.float32),
            pltpu.VMEM_SHARED((n_nodes, _L), jnp.float32),
            pltpu.SemaphoreType.DMA,
            pltpu.SemaphoreType.DMA,
        ],
        compiler_params=_SC_PARAMS,
    )
    def k(dst_hbm, out_hbm, didx0, didx1, ones_v, zb, acc, dsem0, dsem1):
        didxs = (didx0, didx1)
        dsems = (dsem0, dsem1)
        cid = lax.axis_index("c")
        sid = lax.axis_index("s")
        wid = sid * _NC + cid
        base = wid * epw
        one = jnp.ones((_L,), jnp.float32)
        zero = jnp.zeros((_L,), jnp.float32)

        def didx_copy(b, c):
            off = pl.multiple_of(base + c * _CH, 8)
            return pltpu.make_async_copy(
                dst_hbm.at[pl.ds(off, _CH)], didxs[b], dsems[b])

        @pl.loop(0, _CH)
        def _(r):
            ones_v[r, :] = one

        @pl.loop(0, rpt)
        def _(r):
            zb[r, :] = zero

        row0 = sid * rpt
        pltpu.sync_copy(zb, acc.at[pl.ds(row0, rpt)])
        plsc.subcore_barrier()
        for b in range(nb):
            didx_copy(b, b).start()

        @pl.loop(0, nr)
        def _(r):
            for b in range(nb):
                c = r * nb + b
                didx_copy(b, c).wait()
                pltpu.sync_copy(ones_v, acc.at[didxs[b]], add=True)

                @pl.when(r < nr - 1)
                def _():
                    didx_copy(b, c + nb).start()

        plsc.subcore_barrier()
        pltpu.sync_copy(acc.at[pl.ds(row0, rpt)],
                        out_hbm.at[cid, pl.ds(row0, rpt)])

    return k(dst)


@functools.lru_cache(maxsize=None)
def _sc_aggregate_kernel(e, d, n_nodes):
    epw = e // _NW
    nch = epw // _CH
    rpt = n_nodes // _NS
    zr = 125
    nb = 2
    nr = nch // nb

    @functools.partial(
        pl.kernel,
        out_type=jax.ShapeDtypeStruct((_NC, n_nodes, d), jnp.float32),
        mesh=_mesh(),
        scratch_types=[
            pltpu.VMEM((epw,), jnp.int32),
            pltpu.VMEM((_CH,), jnp.int32),
            pltpu.VMEM((_CH,), jnp.int32),
            pltpu.VMEM((_CH, d), jnp.float32),
            pltpu.VMEM((_CH, d), jnp.float32),
            pltpu.VMEM((zr, d), jnp.float32),
            pltpu.VMEM_SHARED((n_nodes, d), jnp.float32),
            pltpu.SemaphoreType.DMA,
            pltpu.SemaphoreType.DMA,
            pltpu.SemaphoreType.DMA,
            pltpu.SemaphoreType.DMA,
        ],
        compiler_params=_SC_PARAMS,
    )
    def k(xs_hbm, src_hbm, dst_hbm, out_hbm, src_all, didx0, didx1,
          rows0, rows1, zb, acc, dsem0, dsem1, gsem0, gsem1):
        didxs = (didx0, didx1)
        rows = (rows0, rows1)
        dsems = (dsem0, dsem1)
        gsems = (gsem0, gsem1)
        cid = lax.axis_index("c")
        sid = lax.axis_index("s")
        wid = sid * _NC + cid
        base = wid * epw
        zero = jnp.zeros((_L,), jnp.float32)

        def didx_copy(b, c):
            off = pl.multiple_of(base + c * _CH, 8)
            return pltpu.make_async_copy(
                dst_hbm.at[pl.ds(off, _CH)], didxs[b], dsems[b])

        def gather_copy(b, c):
            off = pl.multiple_of(c * _CH, 8)
            return pltpu.make_async_copy(
                xs_hbm.at[src_all.at[pl.ds(off, _CH)]], rows[b], gsems[b])

        pltpu.sync_copy(src_hbm.at[pl.ds(pl.multiple_of(base, 8), epw)],
                        src_all)

        @pl.loop(0, zr)
        def _(r):
            @pl.loop(0, d // _L)
            def _(j):
                zb[r, pl.ds(j * _L, _L)] = zero

        row0 = sid * rpt

        @pl.loop(0, rpt // zr)
        def _(t):
            pltpu.sync_copy(zb, acc.at[pl.ds(row0 + t * zr, zr)])

        plsc.subcore_barrier()
        for b in range(nb):
            didx_copy(b, b).start()
            gather_copy(b, b).start()

        @pl.loop(0, nr)
        def _(r):
            for b in range(nb):
                c = r * nb + b
                didx_copy(b, c).wait()
                gather_copy(b, c).wait()
                pltpu.sync_copy(rows[b], acc.at[didxs[b]], add=True)

                @pl.when(r < nr - 1)
                def _():
                    didx_copy(b, c + nb).start()
                    gather_copy(b, c + nb).start()

        plsc.subcore_barrier()

        @pl.loop(0, rpt // zr)
        def _(t):
            r0 = row0 + t * zr
            pltpu.sync_copy(acc.at[pl.ds(r0, zr)],
                            out_hbm.at[cid, pl.ds(r0, zr)])

    return k


def _sc_aggregate(xs, src, dst, n_nodes):
    return _sc_aggregate_kernel(src.shape[0], xs.shape[1], n_nodes)(
        xs, src, dst)


_RB = 1000


def _tc_prep1(x, w1, degp):
    n, d = x.shape

    def body(x_ref, w_ref, degp_ref, xs_ref, dinv_ref):
        deg = degp_ref[0, :, 0:1] + degp_ref[1, :, 0:1] + 1.0
        dinv = lax.rsqrt(deg)
        xw = jnp.dot(x_ref[...], w_ref[...],
                     preferred_element_type=jnp.float32)
        xs_ref[...] = xw * dinv
        dinv_ref[...] = dinv

    return pl.pallas_call(
        body,
        grid=(n // _RB,),
        in_specs=[
            pl.BlockSpec((_RB, d), lambda i: (i, 0)),
            pl.BlockSpec((d, d), lambda i: (0, 0)),
            pl.BlockSpec((_NC, _RB, _L), lambda i: (0, i, 0)),
        ],
        out_specs=[
            pl.BlockSpec((_RB, d), lambda i: (i, 0)),
            pl.BlockSpec((_RB, 1), lambda i: (i, 0)),
        ],
        out_shape=[
            jax.ShapeDtypeStruct((n, d), jnp.float32),
            jax.ShapeDtypeStruct((n, 1), jnp.float32),
        ],
    )(x, w1, degp)


def _tc_mid(agg, xs1, dinv, b1, w2):
    n, d = xs1.shape

    def body(agg_ref, xs_ref, dinv_ref, b_ref, w_ref, o_ref):
        s = agg_ref[0] + agg_ref[1] + xs_ref[...]
        h = jnp.maximum(dinv_ref[...] * s + b_ref[...], 0.0)
        o_ref[...] = jnp.dot(h, w_ref[...],
                             preferred_element_type=jnp.float32) * dinv_ref[...]

    return pl.pallas_call(
        body,
        grid=(n // _RB,),
        in_specs=[
            pl.BlockSpec((_NC, _RB, d), lambda i: (0, i, 0)),
            pl.BlockSpec((_RB, d), lambda i: (i, 0)),
            pl.BlockSpec((_RB, 1), lambda i: (i, 0)),
            pl.BlockSpec((1, d), lambda i: (0, 0)),
            pl.BlockSpec((d, d), lambda i: (0, 0)),
        ],
        out_specs=pl.BlockSpec((_RB, d), lambda i: (i, 0)),
        out_shape=jax.ShapeDtypeStruct((n, d), jnp.float32),
    )(agg, xs1, dinv, b1, w2)


def _tc_final(agg, xs2, dinv, b2):
    n, d = xs2.shape

    def body(agg_ref, xs_ref, dinv_ref, b_ref, o_ref):
        s = agg_ref[0] + agg_ref[1] + xs_ref[...]
        o_ref[...] = dinv_ref[...] * s + b_ref[...]

    return pl.pallas_call(
        body,
        grid=(n // _RB,),
        in_specs=[
            pl.BlockSpec((_NC, _RB, d), lambda i: (0, i, 0)),
            pl.BlockSpec((_RB, d), lambda i: (i, 0)),
            pl.BlockSpec((_RB, 1), lambda i: (i, 0)),
            pl.BlockSpec((1, d), lambda i: (0, 0)),
        ],
        out_specs=pl.BlockSpec((_RB, d), lambda i: (i, 0)),
        out_shape=jax.ShapeDtypeStruct((n, d), jnp.float32),
    )(agg, xs2, dinv, b2)


def kernel(x, adj, W1, b1, W2, b2):
    n, d = x.shape
    adj = adj.astype(jnp.int32)
    src = adj[0]
    dst = adj[1]
    e = src.shape[0]
    assert e % (_NW * _CH) == 0 and n % _NS == 0 and d % _L == 0

    degp = _sc_degree(dst, n)
    xs1, dinv = _tc_prep1(x, W1, degp)
    agg1 = _sc_aggregate(xs1, src, dst, n)
    xs2 = _tc_mid(agg1, xs1, dinv, b1.reshape(1, d), W2)
    agg2 = _sc_aggregate(xs2, src, dst, n)
    return _tc_final(agg2, xs2, dinv, b2.reshape(1, d))


# speedup vs baseline: 29.6885x; 1.3168x over previous
import functools

import jax
import jax.numpy as jnp
from jax import lax
from jax.experimental import pallas as pl
from jax.experimental.pallas import tpu as pltpu
from jax.experimental.pallas import tpu_sc as plsc

_NC = 2
_NS = 16
_L = 16
_NW = _NC * _NS
_CH = 80


def _mesh():
    return plsc.VectorSubcoreMesh(
        core_axis_name="c", subcore_axis_name="s",
        num_cores=_NC, num_subcores=_NS)


_SC_PARAMS = pltpu.CompilerParams(use_tc_tiling_on_sc=False)


def _sc_degree(dst, n_nodes):
    e = dst.shape[0]
    epw = e // _NW
    nch = epw // _CH
    rpt = n_nodes // _NS
    nb = 2
    nr = nch // nb
    tail = list(range(nr * nb, nch))

    @functools.partial(
        pl.kernel,
        out_type=jax.ShapeDtypeStruct((_NC, n_nodes, _L), jnp.float32),
        mesh=_mesh(),
        scratch_types=[
            pltpu.VMEM((_CH,), jnp.int32),
            pltpu.VMEM((_CH,), jnp.int32),
            pltpu.VMEM((_CH, _L), jnp.float32),
            pltpu.VMEM((rpt, _L), jnp.float32),
            pltpu.VMEM_SHARED((n_nodes, _L), jnp.float32),
            pltpu.SemaphoreType.DMA,
            pltpu.SemaphoreType.DMA,
        ],
        compiler_params=_SC_PARAMS,
    )
    def k(dst_hbm, out_hbm, didx0, didx1, ones_v, zb, acc, dsem0, dsem1):
        didxs = (didx0, didx1)
        dsems = (dsem0, dsem1)
        cid = lax.axis_index("c")
        sid = lax.axis_index("s")
        wid = sid * _NC + cid
        base = wid * epw
        one = jnp.ones((_L,), jnp.float32)
        zero = jnp.zeros((_L,), jnp.float32)

        def didx_copy(b, c):
            off = pl.multiple_of(base + c * _CH, 8)
            return pltpu.make_async_copy(
                dst_hbm.at[pl.ds(off, _CH)], didxs[b], dsems[b])

        @pl.loop(0, _CH)
        def _(r):
            ones_v[r, :] = one

        @pl.loop(0, rpt)
        def _(r):
            zb[r, :] = zero

        row0 = sid * rpt
        pltpu.sync_copy(zb, acc.at[pl.ds(row0, rpt)])
        plsc.subcore_barrier()
        for b in range(nb):
            didx_copy(b, b).start()

        @pl.loop(0, nr)
        def _(r):
            for b in range(nb):
                c = r * nb + b
                didx_copy(b, c).wait()
                pltpu.sync_copy(ones_v, acc.at[didxs[b]], add=True)

                @pl.when(c + nb < nch)
                def _():
                    didx_copy(b, c + nb).start()

        for c in tail:
            b = c % nb
            didx_copy(b, c).wait()
            pltpu.sync_copy(ones_v, acc.at[didxs[b]], add=True)

        plsc.subcore_barrier()
        pltpu.sync_copy(acc.at[pl.ds(row0, rpt)],
                        out_hbm.at[cid, pl.ds(row0, rpt)])

    return k(dst)


@functools.lru_cache(maxsize=None)
def _sc_aggregate_kernel(e, d, n_nodes):
    epw = e // _NW
    nch = epw // _CH
    rpt = n_nodes // _NS
    zr = 125
    nb = 2
    nr = nch // nb
    tail = list(range(nr * nb, nch))

    @functools.partial(
        pl.kernel,
        out_type=jax.ShapeDtypeStruct((_NC, n_nodes, d), jnp.float32),
        mesh=_mesh(),
        scratch_types=[
            pltpu.VMEM((epw,), jnp.int32),
            pltpu.VMEM((_CH,), jnp.int32),
            pltpu.VMEM((_CH,), jnp.int32),
            pltpu.VMEM((_CH, d), jnp.float32),
            pltpu.VMEM((_CH, d), jnp.float32),
            pltpu.VMEM((zr, d), jnp.float32),
            pltpu.VMEM_SHARED((n_nodes, d), jnp.float32),
            pltpu.SemaphoreType.DMA,
            pltpu.SemaphoreType.DMA,
            pltpu.SemaphoreType.DMA,
            pltpu.SemaphoreType.DMA,
        ],
        compiler_params=_SC_PARAMS,
    )
    def k(xs_hbm, src_hbm, dst_hbm, out_hbm, src_all, didx0, didx1,
          rows0, rows1, zb, acc, dsem0, dsem1, gsem0, gsem1):
        didxs = (didx0, didx1)
        rows = (rows0, rows1)
        dsems = (dsem0, dsem1)
        gsems = (gsem0, gsem1)
        cid = lax.axis_index("c")
        sid = lax.axis_index("s")
        wid = sid * _NC + cid
        base = wid * epw
        zero = jnp.zeros((_L,), jnp.float32)

        def didx_copy(b, c):
            off = pl.multiple_of(base + c * _CH, 8)
            return pltpu.make_async_copy(
                dst_hbm.at[pl.ds(off, _CH)], didxs[b], dsems[b])

        def gather_copy(b, c):
            off = pl.multiple_of(c * _CH, 8)
            return pltpu.make_async_copy(
                xs_hbm.at[src_all.at[pl.ds(off, _CH)]], rows[b], gsems[b])

        pltpu.sync_copy(src_hbm.at[pl.ds(pl.multiple_of(base, 8), epw)],
                        src_all)

        @pl.loop(0, zr)
        def _(r):
            @pl.loop(0, d // _L)
            def _(j):
                zb[r, pl.ds(j * _L, _L)] = zero

        row0 = sid * rpt

        @pl.loop(0, rpt // zr)
        def _(t):
            pltpu.sync_copy(zb, acc.at[pl.ds(row0 + t * zr, zr)])

        plsc.subcore_barrier()
        for b in range(nb):
            didx_copy(b, b).start()
            gather_copy(b, b).start()

        @pl.loop(0, nr)
        def _(r):
            for b in range(nb):
                c = r * nb + b
                didx_copy(b, c).wait()
                gather_copy(b, c).wait()
                pltpu.sync_copy(rows[b], acc.at[didxs[b]], add=True)

                @pl.when(c + nb < nch)
                def _():
                    didx_copy(b, c + nb).start()
                    gather_copy(b, c + nb).start()

        for c in tail:
            b = c % nb
            didx_copy(b, c).wait()
            gather_copy(b, c).wait()
            pltpu.sync_copy(rows[b], acc.at[didxs[b]], add=True)

        plsc.subcore_barrier()

        @pl.loop(0, rpt // zr)
        def _(t):
            r0 = row0 + t * zr
            pltpu.sync_copy(acc.at[pl.ds(r0, zr)],
                            out_hbm.at[cid, pl.ds(r0, zr)])

    return k


def _sc_aggregate(xs, src, dst, n_nodes):
    return _sc_aggregate_kernel(src.shape[0], xs.shape[1], n_nodes)(
        xs, src, dst)


_RB = 1000


def _tc_prep1(x, w1, degp):
    n, d = x.shape

    def body(x_ref, w_ref, degp_ref, xs_ref, dinv_ref):
        deg = degp_ref[0, :, 0:1] + degp_ref[1, :, 0:1] + 1.0
        dinv = lax.rsqrt(deg)
        xw = jnp.dot(x_ref[...], w_ref[...],
                     preferred_element_type=jnp.float32)
        xs_ref[...] = xw * dinv
        dinv_ref[...] = dinv

    return pl.pallas_call(
        body,
        grid=(n // _RB,),
        in_specs=[
            pl.BlockSpec((_RB, d), lambda i: (i, 0)),
            pl.BlockSpec((d, d), lambda i: (0, 0)),
            pl.BlockSpec((_NC, _RB, _L), lambda i: (0, i, 0)),
        ],
        out_specs=[
            pl.BlockSpec((_RB, d), lambda i: (i, 0)),
            pl.BlockSpec((_RB, 1), lambda i: (i, 0)),
        ],
        out_shape=[
            jax.ShapeDtypeStruct((n, d), jnp.float32),
            jax.ShapeDtypeStruct((n, 1), jnp.float32),
        ],
    )(x, w1, degp)


def _tc_mid(agg, xs1, dinv, b1, w2):
    n, d = xs1.shape

    def body(agg_ref, xs_ref, dinv_ref, b_ref, w_ref, o_ref):
        s = agg_ref[0] + agg_ref[1] + xs_ref[...]
        h = jnp.maximum(dinv_ref[...] * s + b_ref[...], 0.0)
        o_ref[...] = jnp.dot(h, w_ref[...],
                             preferred_element_type=jnp.float32) * dinv_ref[...]

    return pl.pallas_call(
        body,
        grid=(n // _RB,),
        in_specs=[
            pl.BlockSpec((_NC, _RB, d), lambda i: (0, i, 0)),
            pl.BlockSpec((_RB, d), lambda i: (i, 0)),
            pl.BlockSpec((_RB, 1), lambda i: (i, 0)),
            pl.BlockSpec((1, d), lambda i: (0, 0)),
            pl.BlockSpec((d, d), lambda i: (0, 0)),
        ],
        out_specs=pl.BlockSpec((_RB, d), lambda i: (i, 0)),
        out_shape=jax.ShapeDtypeStruct((n, d), jnp.float32),
    )(agg, xs1, dinv, b1, w2)


def _tc_final(agg, xs2, dinv, b2):
    n, d = xs2.shape

    def body(agg_ref, xs_ref, dinv_ref, b_ref, o_ref):
        s = agg_ref[0] + agg_ref[1] + xs_ref[...]
        o_ref[...] = dinv_ref[...] * s + b_ref[...]

    return pl.pallas_call(
        body,
        grid=(n // _RB,),
        in_specs=[
            pl.BlockSpec((_NC, _RB, d), lambda i: (0, i, 0)),
            pl.BlockSpec((_RB, d), lambda i: (i, 0)),
            pl.BlockSpec((_RB, 1), lambda i: (i, 0)),
            pl.BlockSpec((1, d), lambda i: (0, 0)),
        ],
        out_specs=pl.BlockSpec((_RB, d), lambda i: (i, 0)),
        out_shape=jax.ShapeDtypeStruct((n, d), jnp.float32),
    )(agg, xs2, dinv, b2)


def kernel(x, adj, W1, b1, W2, b2):
    n, d = x.shape
    adj = adj.astype(jnp.int32)
    src = adj[0]
    dst = adj[1]
    e = src.shape[0]
    assert e % (_NW * _CH) == 0 and n % _NS == 0 and d % _L == 0

    degp = _sc_degree(dst, n)
    xs1, dinv = _tc_prep1(x, W1, degp)
    agg1 = _sc_aggregate(xs1, src, dst, n)
    xs2 = _tc_mid(agg1, xs1, dinv, b1.reshape(1, d), W2)
    agg2 = _sc_aggregate(xs2, src, dst, n)
    return _tc_final(agg2, xs2, dinv, b2.reshape(1, d))


# speedup vs baseline: 30.3746x; 1.0231x over previous
import functools

import jax
import jax.numpy as jnp
from jax import lax
from jax.experimental import pallas as pl
from jax.experimental.pallas import tpu as pltpu
from jax.experimental.pallas import tpu_sc as plsc

_NC = 2
_NS = 16
_L = 16
_NW = _NC * _NS
_CHA = 80
_CHD = 80


def _mesh():
    return plsc.VectorSubcoreMesh(
        core_axis_name="c", subcore_axis_name="s",
        num_cores=_NC, num_subcores=_NS)


_SC_PARAMS = pltpu.CompilerParams(use_tc_tiling_on_sc=False)


def _sc_degree(dst, n_nodes):
    e = dst.shape[0]
    epw = e // _NW
    nch = epw // _CHD
    nb = 2
    nr = nch // nb
    tail = list(range(nr * nb, nch))
    fl = 640
    assert fl % 8 == 0 and (_NS - 1) * fl < n_nodes

    @functools.partial(
        pl.kernel,
        out_type=jax.ShapeDtypeStruct((_NC, n_nodes), jnp.float32),
        mesh=_mesh(),
        scratch_types=[
            pltpu.VMEM((_CHD,), jnp.int32),
            pltpu.VMEM((_CHD,), jnp.int32),
            pltpu.VMEM((_CHD,), jnp.float32),
            pltpu.VMEM((fl,), jnp.float32),
            pltpu.VMEM_SHARED((n_nodes,), jnp.float32),
            pltpu.SemaphoreType.DMA,
            pltpu.SemaphoreType.DMA,
        ],
        compiler_params=_SC_PARAMS,
    )
    def k(dst_hbm, out_hbm, didx0, didx1, ones_v, zb, acc, dsem0, dsem1):
        didxs = (didx0, didx1)
        dsems = (dsem0, dsem1)
        cid = lax.axis_index("c")
        sid = lax.axis_index("s")
        wid = sid * _NC + cid
        base = wid * epw
        one = jnp.ones((_L,), jnp.float32)
        zero = jnp.zeros((_L,), jnp.float32)

        def didx_copy(b, c):
            off = pl.multiple_of(base + c * _CHD, 8)
            return pltpu.make_async_copy(
                dst_hbm.at[pl.ds(off, _CHD)], didxs[b], dsems[b])

        @pl.loop(0, _CHD // _L)
        def _(r):
            ones_v[pl.ds(r * _L, _L)] = one

        @pl.loop(0, fl // _L)
        def _(r):
            zb[pl.ds(r * _L, _L)] = zero

        lastn = n_nodes - (_NS - 1) * fl
        row0 = sid * fl
        pltpu.sync_copy(zb.at[pl.ds(0, lastn)], acc.at[pl.ds(row0, lastn)])

        @pl.when(sid < _NS - 1)
        def _():
            pltpu.sync_copy(zb.at[pl.ds(0, fl - lastn)],
                            acc.at[pl.ds(row0 + lastn, fl - lastn)])

        plsc.subcore_barrier()
        for b in range(nb):
            didx_copy(b, b).start()

        @pl.loop(0, nr)
        def _(r):
            for b in range(nb):
                c = r * nb + b
                didx_copy(b, c).wait()
                pltpu.sync_copy(ones_v, acc.at[didxs[b]], add=True)

                @pl.when(c + nb < nch)
                def _():
                    didx_copy(b, c + nb).start()

        for c in tail:
            b = c % nb
            didx_copy(b, c).wait()
            pltpu.sync_copy(ones_v, acc.at[didxs[b]], add=True)

        plsc.subcore_barrier()
        pltpu.sync_copy(acc.at[pl.ds(row0, lastn)],
                        out_hbm.at[cid, pl.ds(row0, lastn)])

        @pl.when(sid < _NS - 1)
        def _():
            pltpu.sync_copy(acc.at[pl.ds(row0 + lastn, fl - lastn)],
                            out_hbm.at[cid, pl.ds(row0 + lastn, fl - lastn)])

    return k(dst)


@functools.lru_cache(maxsize=None)
def _sc_aggregate_kernel(e, d, n_nodes):
    epw = e // _NW
    nch = epw // _CHA
    rpt = n_nodes // _NS
    zr = 125
    nb = 2
    nr = nch // nb
    tail = list(range(nr * nb, nch))

    @functools.partial(
        pl.kernel,
        out_type=jax.ShapeDtypeStruct((_NC, n_nodes, d), jnp.float32),
        mesh=_mesh(),
        scratch_types=[
            pltpu.VMEM((epw,), jnp.int32),
            pltpu.VMEM((_CHA,), jnp.int32),
            pltpu.VMEM((_CHA,), jnp.int32),
            pltpu.VMEM((_CHA, d), jnp.float32),
            pltpu.VMEM((_CHA, d), jnp.float32),
            pltpu.VMEM((zr, d), jnp.float32),
            pltpu.VMEM_SHARED((n_nodes, d), jnp.float32),
            pltpu.SemaphoreType.DMA,
            pltpu.SemaphoreType.DMA,
            pltpu.SemaphoreType.DMA,
            pltpu.SemaphoreType.DMA,
        ],
        compiler_params=_SC_PARAMS,
    )
    def k(xs_hbm, src_hbm, dst_hbm, out_hbm, src_all, didx0, didx1,
          rows0, rows1, zb, acc, dsem0, dsem1, gsem0, gsem1):
        didxs = (didx0, didx1)
        rows = (rows0, rows1)
        dsems = (dsem0, dsem1)
        gsems = (gsem0, gsem1)
        cid = lax.axis_index("c")
        sid = lax.axis_index("s")
        wid = sid * _NC + cid
        base = wid * epw
        zero = jnp.zeros((_L,), jnp.float32)

        def didx_copy(b, c):
            off = pl.multiple_of(base + c * _CHA, 8)
            return pltpu.make_async_copy(
                dst_hbm.at[pl.ds(off, _CHA)], didxs[b], dsems[b])

        def gather_copy(b, c):
            off = pl.multiple_of(c * _CHA, 8)
            return pltpu.make_async_copy(
                xs_hbm.at[src_all.at[pl.ds(off, _CHA)]], rows[b], gsems[b])

        pltpu.sync_copy(src_hbm.at[pl.ds(pl.multiple_of(base, 8), epw)],
                        src_all)

        @pl.loop(0, zr)
        def _(r):
            @pl.loop(0, d // _L)
            def _(j):
                zb[r, pl.ds(j * _L, _L)] = zero

        row0 = sid * rpt

        @pl.loop(0, rpt // zr)
        def _(t):
            pltpu.sync_copy(zb, acc.at[pl.ds(row0 + t * zr, zr)])

        plsc.subcore_barrier()
        for b in range(nb):
            didx_copy(b, b).start()
            gather_copy(b, b).start()

        @pl.loop(0, nr)
        def _(r):
            for b in range(nb):
                c = r * nb + b
                didx_copy(b, c).wait()
                gather_copy(b, c).wait()
                pltpu.sync_copy(rows[b], acc.at[didxs[b]], add=True)

                @pl.when(c + nb < nch)
                def _():
                    didx_copy(b, c + nb).start()
                    gather_copy(b, c + nb).start()

        for c in tail:
            b = c % nb
            didx_copy(b, c).wait()
            gather_copy(b, c).wait()
            pltpu.sync_copy(rows[b], acc.at[didxs[b]], add=True)

        plsc.subcore_barrier()

        @pl.loop(0, rpt // zr)
        def _(t):
            r0 = row0 + t * zr
            pltpu.sync_copy(acc.at[pl.ds(r0, zr)],
                            out_hbm.at[cid, pl.ds(r0, zr)])

    return k


def _sc_aggregate(xs, src, dst, n_nodes):
    return _sc_aggregate_kernel(src.shape[0], xs.shape[1], n_nodes)(
        xs, src, dst)


_RB = 1000


def _tc_prep1(x, w1, degp):
    n, d = x.shape

    def body(x_ref, w_ref, degp_ref, xs_ref, dinv_ref):
        deg = degp_ref[:, 0:1] + degp_ref[:, 1:2] + 1.0
        dinv = lax.rsqrt(deg)
        xw = jnp.dot(x_ref[...], w_ref[...],
                     preferred_element_type=jnp.float32)
        xs_ref[...] = xw * dinv
        dinv_ref[...] = dinv

    return pl.pallas_call(
        body,
        grid=(n // _RB,),
        in_specs=[
            pl.BlockSpec((_RB, d), lambda i: (i, 0)),
            pl.BlockSpec((d, d), lambda i: (0, 0)),
            pl.BlockSpec((_RB, _NC), lambda i: (i, 0)),
        ],
        out_specs=[
            pl.BlockSpec((_RB, d), lambda i: (i, 0)),
            pl.BlockSpec((_RB, 1), lambda i: (i, 0)),
        ],
        out_shape=[
            jax.ShapeDtypeStruct((n, d), jnp.float32),
            jax.ShapeDtypeStruct((n, 1), jnp.float32),
        ],
    )(x, w1, degp)


def _tc_mid(agg, xs1, dinv, b1, w2):
    n, d = xs1.shape

    def body(agg_ref, xs_ref, dinv_ref, b_ref, w_ref, o_ref):
        s = agg_ref[0] + agg_ref[1] + xs_ref[...]
        h = jnp.maximum(dinv_ref[...] * s + b_ref[...], 0.0)
        o_ref[...] = jnp.dot(h, w_ref[...],
                             preferred_element_type=jnp.float32) * dinv_ref[...]

    return pl.pallas_call(
        body,
        grid=(n // _RB,),
        in_specs=[
            pl.BlockSpec((_NC, _RB, d), lambda i: (0, i, 0)),
            pl.BlockSpec((_RB, d), lambda i: (i, 0)),
            pl.BlockSpec((_RB, 1), lambda i: (i, 0)),
            pl.BlockSpec((1, d), lambda i: (0, 0)),
            pl.BlockSpec((d, d), lambda i: (0, 0)),
        ],
        out_specs=pl.BlockSpec((_RB, d), lambda i: (i, 0)),
        out_shape=jax.ShapeDtypeStruct((n, d), jnp.float32),
    )(agg, xs1, dinv, b1, w2)


def _tc_final(agg, xs2, dinv, b2):
    n, d = xs2.shape

    def body(agg_ref, xs_ref, dinv_ref, b_ref, o_ref):
        s = agg_ref[0] + agg_ref[1] + xs_ref[...]
        o_ref[...] = dinv_ref[...] * s + b_ref[...]

    return pl.pallas_call(
        body,
        grid=(n // _RB,),
        in_specs=[
            pl.BlockSpec((_NC, _RB, d), lambda i: (0, i, 0)),
            pl.BlockSpec((_RB, d), lambda i: (i, 0)),
            pl.BlockSpec((_RB, 1), lambda i: (i, 0)),
            pl.BlockSpec((1, d), lambda i: (0, 0)),
        ],
        out_specs=pl.BlockSpec((_RB, d), lambda i: (i, 0)),
        out_shape=jax.ShapeDtypeStruct((n, d), jnp.float32),
    )(agg, xs2, dinv, b2)


def kernel(x, adj, W1, b1, W2, b2):
    n, d = x.shape
    adj = adj.astype(jnp.int32)
    src = adj[0]
    dst = adj[1]
    e = src.shape[0]
    assert e % _NW == 0 and n % _NS == 0 and d % _L == 0

    degp = _sc_degree(dst, n)
    degp_t = jnp.transpose(degp)
    xs1, dinv = _tc_prep1(x, W1, degp_t)
    agg1 = _sc_aggregate(xs1, src, dst, n)
    xs2 = _tc_mid(agg1, xs1, dinv, b1.reshape(1, d), W2)
    agg2 = _sc_aggregate(xs2, src, dst, n)
    return _tc_final(agg2, xs2, dinv, b2.reshape(1, d))


# speedup vs baseline: 34.9677x; 1.1512x over previous
import functools

import jax
import jax.numpy as jnp
from jax import lax
from jax.experimental import pallas as pl
from jax.experimental.pallas import tpu as pltpu
from jax.experimental.pallas import tpu_sc as plsc

_NC = 2
_NS = 16
_L = 16
_NW = _NC * _NS
_CHA = 80
_CHD = 80


def _mesh():
    return plsc.VectorSubcoreMesh(
        core_axis_name="c", subcore_axis_name="s",
        num_cores=_NC, num_subcores=_NS)


_SC_PARAMS = pltpu.CompilerParams(use_tc_tiling_on_sc=False)


def _sc_degree(dst, n_nodes):
    e = dst.shape[0]
    epw = e // _NW
    nch = epw // _CHD
    nb = 4
    nr = nch // nb
    tail = list(range(nr * nb, nch))
    fl = 640
    assert fl % 8 == 0 and (_NS - 1) * fl < n_nodes

    @functools.partial(
        pl.kernel,
        out_type=jax.ShapeDtypeStruct((_NC, n_nodes), jnp.float32),
        mesh=_mesh(),
        scratch_types=[
            pltpu.VMEM((_CHD,), jnp.int32),
            pltpu.VMEM((_CHD,), jnp.int32),
            pltpu.VMEM((_CHD,), jnp.int32),
            pltpu.VMEM((_CHD,), jnp.int32),
            pltpu.VMEM((_CHD,), jnp.float32),
            pltpu.VMEM((fl,), jnp.float32),
            pltpu.VMEM_SHARED((n_nodes,), jnp.float32),
            pltpu.SemaphoreType.DMA,
            pltpu.SemaphoreType.DMA,
            pltpu.SemaphoreType.DMA,
            pltpu.SemaphoreType.DMA,
        ],
        compiler_params=_SC_PARAMS,
    )
    def k(dst_hbm, out_hbm, didx0, didx1, didx2, didx3, ones_v, zb, acc,
          dsem0, dsem1, dsem2, dsem3):
        didxs = (didx0, didx1, didx2, didx3)
        dsems = (dsem0, dsem1, dsem2, dsem3)
        cid = lax.axis_index("c")
        sid = lax.axis_index("s")
        wid = sid * _NC + cid
        base = wid * epw
        one = jnp.ones((_L,), jnp.float32)
        zero = jnp.zeros((_L,), jnp.float32)

        def didx_copy(b, c):
            off = pl.multiple_of(base + c * _CHD, 8)
            return pltpu.make_async_copy(
                dst_hbm.at[pl.ds(off, _CHD)], didxs[b], dsems[b])

        @pl.loop(0, _CHD // _L)
        def _(r):
            ones_v[pl.ds(r * _L, _L)] = one

        @pl.loop(0, fl // _L)
        def _(r):
            zb[pl.ds(r * _L, _L)] = zero

        lastn = n_nodes - (_NS - 1) * fl
        row0 = sid * fl
        pltpu.sync_copy(zb.at[pl.ds(0, lastn)], acc.at[pl.ds(row0, lastn)])

        @pl.when(sid < _NS - 1)
        def _():
            pltpu.sync_copy(zb.at[pl.ds(0, fl - lastn)],
                            acc.at[pl.ds(row0 + lastn, fl - lastn)])

        plsc.subcore_barrier()
        for b in range(nb):
            didx_copy(b, b).start()

        @pl.loop(0, nr)
        def _(r):
            for b in range(nb):
                c = r * nb + b
                didx_copy(b, c).wait()
                pltpu.sync_copy(ones_v, acc.at[didxs[b]], add=True)

                @pl.when(c + nb < nch)
                def _():
                    didx_copy(b, c + nb).start()

        for c in tail:
            b = c % nb
            didx_copy(b, c).wait()
            pltpu.sync_copy(ones_v, acc.at[didxs[b]], add=True)

        plsc.subcore_barrier()
        pltpu.sync_copy(acc.at[pl.ds(row0, lastn)],
                        out_hbm.at[cid, pl.ds(row0, lastn)])

        @pl.when(sid < _NS - 1)
        def _():
            pltpu.sync_copy(acc.at[pl.ds(row0 + lastn, fl - lastn)],
                            out_hbm.at[cid, pl.ds(row0 + lastn, fl - lastn)])

    return k(dst)


@functools.lru_cache(maxsize=None)
def _sc_aggregate_kernel(e, d, n_nodes):
    epw = e // _NW
    nch = epw // _CHA
    rpt = n_nodes // _NS
    zr = 25
    nb = 3
    nr = nch // nb
    tail = list(range(nr * nb, nch))

    @functools.partial(
        pl.kernel,
        out_type=jax.ShapeDtypeStruct((_NC, n_nodes, d), jnp.float32),
        mesh=_mesh(),
        scratch_types=[
            pltpu.VMEM((epw,), jnp.int32),
            pltpu.VMEM((_CHA,), jnp.int32),
            pltpu.VMEM((_CHA,), jnp.int32),
            pltpu.VMEM((_CHA,), jnp.int32),
            pltpu.VMEM((_CHA, d), jnp.float32),
            pltpu.VMEM((_CHA, d), jnp.float32),
            pltpu.VMEM((_CHA, d), jnp.float32),
            pltpu.VMEM((zr, d), jnp.float32),
            pltpu.VMEM_SHARED((n_nodes, d), jnp.float32),
            pltpu.SemaphoreType.DMA,
            pltpu.SemaphoreType.DMA,
            pltpu.SemaphoreType.DMA,
            pltpu.SemaphoreType.DMA,
            pltpu.SemaphoreType.DMA,
            pltpu.SemaphoreType.DMA,
        ],
        compiler_params=_SC_PARAMS,
    )
    def k(xs_hbm, src_hbm, dst_hbm, out_hbm, src_all, didx0, didx1, didx2,
          rows0, rows1, rows2, zb, acc, dsem0, dsem1, dsem2,
          gsem0, gsem1, gsem2):
        didxs = (didx0, didx1, didx2)
        rows = (rows0, rows1, rows2)
        dsems = (dsem0, dsem1, dsem2)
        gsems = (gsem0, gsem1, gsem2)
        cid = lax.axis_index("c")
        sid = lax.axis_index("s")
        wid = sid * _NC + cid
        base = wid * epw
        zero = jnp.zeros((_L,), jnp.float32)

        def didx_copy(b, c):
            off = pl.multiple_of(base + c * _CHA, 8)
            return pltpu.make_async_copy(
                dst_hbm.at[pl.ds(off, _CHA)], didxs[b], dsems[b])

        def gather_copy(b, c):
            off = pl.multiple_of(c * _CHA, 8)
            return pltpu.make_async_copy(
                xs_hbm.at[src_all.at[pl.ds(off, _CHA)]], rows[b], gsems[b])

        pltpu.sync_copy(src_hbm.at[pl.ds(pl.multiple_of(base, 8), epw)],
                        src_all)

        @pl.loop(0, zr)
        def _(r):
            @pl.loop(0, d // _L)
            def _(j):
                zb[r, pl.ds(j * _L, _L)] = zero

        row0 = sid * rpt

        @pl.loop(0, rpt // zr)
        def _(t):
            pltpu.sync_copy(zb, acc.at[pl.ds(row0 + t * zr, zr)])

        plsc.subcore_barrier()
        for b in range(nb):
            didx_copy(b, b).start()
            gather_copy(b, b).start()

        @pl.loop(0, nr)
        def _(r):
            for b in range(nb):
                c = r * nb + b
                didx_copy(b, c).wait()
                gather_copy(b, c).wait()
                pltpu.sync_copy(rows[b], acc.at[didxs[b]], add=True)

                @pl.when(c + nb < nch)
                def _():
                    didx_copy(b, c + nb).start()
                    gather_copy(b, c + nb).start()

        for c in tail:
            b = c % nb
            didx_copy(b, c).wait()
            gather_copy(b, c).wait()
            pltpu.sync_copy(rows[b], acc.at[didxs[b]], add=True)

        plsc.subcore_barrier()

        @pl.loop(0, rpt // zr)
        def _(t):
            r0 = row0 + t * zr
            pltpu.sync_copy(acc.at[pl.ds(r0, zr)],
                            out_hbm.at[cid, pl.ds(r0, zr)])

    return k


def _sc_aggregate(xs, src, dst, n_nodes):
    return _sc_aggregate_kernel(src.shape[0], xs.shape[1], n_nodes)(
        xs, src, dst)


_RB = 2000


def _tc_prep1(x, w1, degp):
    n, d = x.shape

    def body(x_ref, w_ref, degp_ref, xs_ref, dinv_ref):
        deg = degp_ref[:, 0:1] + degp_ref[:, 1:2] + 1.0
        dinv = lax.rsqrt(deg)
        xw = jnp.dot(x_ref[...], w_ref[...],
                     preferred_element_type=jnp.float32)
        xs_ref[...] = xw * dinv
        dinv_ref[...] = dinv

    return pl.pallas_call(
        body,
        grid=(n // _RB,),
        in_specs=[
            pl.BlockSpec((_RB, d), lambda i: (i, 0)),
            pl.BlockSpec((d, d), lambda i: (0, 0)),
            pl.BlockSpec((_RB, _NC), lambda i: (i, 0)),
        ],
        out_specs=[
            pl.BlockSpec((_RB, d), lambda i: (i, 0)),
            pl.BlockSpec((_RB, 1), lambda i: (i, 0)),
        ],
        out_shape=[
            jax.ShapeDtypeStruct((n, d), jnp.float32),
            jax.ShapeDtypeStruct((n, 1), jnp.float32),
        ],
    )(x, w1, degp)


def _tc_mid(agg, xs1, dinv, b1, w2):
    n, d = xs1.shape

    def body(agg_ref, xs_ref, dinv_ref, b_ref, w_ref, o_ref):
        s = agg_ref[0] + agg_ref[1] + xs_ref[...]
        h = jnp.maximum(dinv_ref[...] * s + b_ref[...], 0.0)
        o_ref[...] = jnp.dot(h, w_ref[...],
                             preferred_element_type=jnp.float32) * dinv_ref[...]

    return pl.pallas_call(
        body,
        grid=(n // _RB,),
        in_specs=[
            pl.BlockSpec((_NC, _RB, d), lambda i: (0, i, 0)),
            pl.BlockSpec((_RB, d), lambda i: (i, 0)),
            pl.BlockSpec((_RB, 1), lambda i: (i, 0)),
            pl.BlockSpec((1, d), lambda i: (0, 0)),
            pl.BlockSpec((d, d), lambda i: (0, 0)),
        ],
        out_specs=pl.BlockSpec((_RB, d), lambda i: (i, 0)),
        out_shape=jax.ShapeDtypeStruct((n, d), jnp.float32),
    )(agg, xs1, dinv, b1, w2)


def _tc_final(agg, xs2, dinv, b2):
    n, d = xs2.shape

    def body(agg_ref, xs_ref, dinv_ref, b_ref, o_ref):
        s = agg_ref[0] + agg_ref[1] + xs_ref[...]
        o_ref[...] = dinv_ref[...] * s + b_ref[...]

    return pl.pallas_call(
        body,
        grid=(n // _RB,),
        in_specs=[
            pl.BlockSpec((_NC, _RB, d), lambda i: (0, i, 0)),
            pl.BlockSpec((_RB, d), lambda i: (i, 0)),
            pl.BlockSpec((_RB, 1), lambda i: (i, 0)),
            pl.BlockSpec((1, d), lambda i: (0, 0)),
        ],
        out_specs=pl.BlockSpec((_RB, d), lambda i: (i, 0)),
        out_shape=jax.ShapeDtypeStruct((n, d), jnp.float32),
    )(agg, xs2, dinv, b2)


def kernel(x, adj, W1, b1, W2, b2):
    n, d = x.shape
    adj = adj.astype(jnp.int32)
    src = adj[0]
    dst = adj[1]
    e = src.shape[0]
    assert e % _NW == 0 and n % _NS == 0 and d % _L == 0

    degp = _sc_degree(dst, n)
    degp_t = jnp.transpose(degp)
    xs1, dinv = _tc_prep1(x, W1, degp_t)
    agg1 = _sc_aggregate(xs1, src, dst, n)
    xs2 = _tc_mid(agg1, xs1, dinv, b1.reshape(1, d), W2)
    agg2 = _sc_aggregate(xs2, src, dst, n)
    return _tc_final(agg2, xs2, dinv, b2.reshape(1, d))
